# baseline (speedup 1.0000x reference)
import functools

import jax
import jax.numpy as jnp
from jax import lax
from jax.experimental import pallas as pl
from jax.experimental.pallas import tpu as pltpu
from jax.experimental.pallas import tpu_sc as plsc

K = 32
N = 50000
D = 256
G = 64
BR = 1024
NB = (N + BR - 1) // BR
NPAD = NB * BR
WROWS = 8
WCOLS = 512
W = WROWS * WCOLS
KROWS = 106
KPAD = KROWS * WCOLS
SE_PAD = 160
L = 16
BIGI = 2**30


def _tc_kernel(x_ref, b_ref, keys_ref, se_ref, cnt_ref):
    i = pl.program_id(0)
    x = x_ref[...]
    s = jnp.sum(x * x, axis=1)
    keys_ref[0, 0, :] = jnp.sqrt(s)

    @pl.when(i == 0)
    def _init():
        cnt_ref[...] = jnp.zeros_like(cnt_ref)

    b = b_ref[0, 0, :]
    ids = lax.broadcasted_iota(jnp.int32, (G, BR), 0)
    onehot = (b[None, :] == ids).astype(jnp.float32)
    cnt_ref[...] += jnp.sum(onehot, axis=1)[None, :]

    @pl.when(i == NB - 1)
    def _fin():
        c = cnt_ref[...]
        lane = lax.broadcasted_iota(jnp.int32, (1, G), 1)
        incl = c
        sh = 1
        while sh < G:
            rolled = pltpu.roll(incl, sh, 1)
            incl = incl + jnp.where(lane >= sh, rolled, 0.0)
            sh *= 2
        starts = incl - c
        ends = incl
        se_ref[0, :] = jnp.concatenate(
            [starts[0, :], ends[0, :], jnp.zeros((SE_PAD - 2 * G,))]
        ).astype(jnp.int32)


def _tc_stage(x, batch_pad):
    keys, se = pl.pallas_call(
        _tc_kernel,
        grid=(NB,),
        in_specs=[
            pl.BlockSpec((BR, D), lambda i: (i, 0)),
            pl.BlockSpec((1, 1, BR), lambda i: (i, 0, 0)),
        ],
        out_specs=[
            pl.BlockSpec((1, 1, BR), lambda i: (i, 0, 0)),
            pl.BlockSpec((1, SE_PAD), lambda i: (0, 0)),
        ],
        out_shape=[
            jax.ShapeDtypeStruct((NB, 1, BR), jnp.float32),
            jax.ShapeDtypeStruct((1, SE_PAD), jnp.int32),
        ],
        scratch_shapes=[pltpu.VMEM((1, G), jnp.float32)],
    )(x, batch_pad)
    return keys, se


def _sel_kernel(se_ref, keys_ref, idx_ref, val_ref):
    g = pl.program_id(0)
    seg_s = se_ref[0, g]
    seg_e = se_ref[0, G + g]
    r0 = pl.multiple_of((seg_s // W) * WROWS, WROWS)
    nch = jnp.where(
        seg_e > seg_s, (seg_e - r0 * WCOLS + (W - 1)) // W, 0
    )

    lane32 = lax.broadcasted_iota(jnp.int32, (1, K), 1)
    neg_inf = jnp.float32(-jnp.inf)

    def chunk_body(c, carry):
        best_v, best_i = carry
        wb = (r0 + c * WROWS) * WCOLS
        win = keys_ref[pl.ds(r0 + c * WROWS, WROWS), :]
        gidx = (
            wb
            + lax.broadcasted_iota(jnp.int32, (WROWS, WCOLS), 0) * WCOLS
            + lax.broadcasted_iota(jnp.int32, (WROWS, WCOLS), 1)
        )
        inseg = (gidx >= seg_s) & (gidx < seg_e)
        wm = jnp.where(inseg, win, neg_inf)

        ct_v = jnp.full((1, K), neg_inf)
        ct_i = BIGI + K + lane32
        for t in range(K):
            mval = jnp.max(wm)
            fidx = jnp.min(jnp.where(wm == mval, gidx, BIGI))
            ct_v = jnp.where(lane32 == t, mval, ct_v)
            ct_i = jnp.where(
                lane32 == t,
                jnp.where(mval > neg_inf, fidx, BIGI + K + t),
                ct_i,
            )
            wm = jnp.where(gidx == fidx, neg_inf, wm)

        comb_v = jnp.concatenate([best_v, ct_v], axis=1)
        comb_i = jnp.concatenate([best_i, ct_i], axis=1)
        nb_v = jnp.full((1, K), neg_inf)
        nb_i = BIGI + 2 * K + lane32
        for t in range(K):
            mval = jnp.max(comb_v)
            fidx = jnp.min(jnp.where(comb_v == mval, comb_i, 2**31 - 1))
            nb_v = jnp.where(lane32 == t, mval, nb_v)
            nb_i = jnp.where(lane32 == t, fidx, nb_i)
            comb_v = jnp.where(comb_i == fidx, neg_inf, comb_v)
        return (nb_v, nb_i)

    init = (jnp.full((1, K), neg_inf), BIGI + lane32)
    best_v, best_i = lax.fori_loop(0, nch, chunk_body, init)
    valid = (best_v > neg_inf).astype(jnp.int32)
    idx_ref[0, 0, :] = jnp.where(valid > 0, best_i, 0)[0, :]
    val_ref[0, 0, :] = valid[0, :]


def _sel_stage(se, keys_flat2d):
    idx, val = pl.pallas_call(
        _sel_kernel,
        grid=(G,),
        in_specs=[
            pl.BlockSpec(memory_space=pltpu.SMEM),
            pl.BlockSpec((KROWS, WCOLS), lambda g: (0, 0)),
        ],
        out_specs=[
            pl.BlockSpec((1, 1, K), lambda g: (g, 0, 0)),
            pl.BlockSpec((1, 1, K), lambda g: (g, 0, 0)),
        ],
        out_shape=[
            jax.ShapeDtypeStruct((G, 1, K), jnp.int32),
            jax.ShapeDtypeStruct((G, 1, K), jnp.int32),
        ],
    )(se, keys_flat2d)
    return idx.reshape(-1), val.reshape(-1)


def _sc_body(idx_hbm, val_hbm, x_hbm, out_hbm,
             idx0, idx1, val_v, rows0, rows1, sem):
    wid = lax.axis_index("s") * 2 + lax.axis_index("c")
    base = wid * 2 * K
    pltpu.sync_copy(idx_hbm.at[pl.ds(base, K)], idx0)
    pltpu.sync_copy(idx_hbm.at[pl.ds(base + K, K)], idx1)
    pltpu.sync_copy(val_hbm.at[pl.ds(base, 2 * K)], val_v.at[pl.ds(0, 2 * K)])
    pltpu.async_copy(x_hbm.at[idx0], rows0, sem).wait()
    pltpu.async_copy(x_hbm.at[idx1], rows1, sem).wait()

    def scal(ref, i):
        return ref[pl.ds(i, L)][0]

    rows_refs = (rows0, rows1)
    for gl in range(2):
        def zero_body(r, carry, gl=gl):
            @pl.when(scal(val_v, r + K * gl) == 0)
            def _z():
                for c in range(D // L):
                    rows_refs[gl][r, pl.ds(c * L, L)] = jnp.zeros(
                        (L,), jnp.float32
                    )
            return carry

        lax.fori_loop(0, K, zero_body, 0)

    pltpu.sync_copy(rows0, out_hbm.at[2 * wid])
    pltpu.sync_copy(rows1, out_hbm.at[2 * wid + 1])


def _sc_stage(idx, val, x):
    mesh = plsc.VectorSubcoreMesh(core_axis_name="c", subcore_axis_name="s")
    run = functools.partial(
        pl.kernel,
        mesh=mesh,
        compiler_params=pltpu.CompilerParams(use_tc_tiling_on_sc=False),
        out_type=jax.ShapeDtypeStruct((G, K, D), jnp.float32),
        scratch_types=[
            pltpu.VMEM((K,), jnp.int32),
            pltpu.VMEM((K,), jnp.int32),
            pltpu.VMEM((2 * K + L,), jnp.int32),
            pltpu.VMEM((K, D), jnp.float32),
            pltpu.VMEM((K, D), jnp.float32),
            pltpu.SemaphoreType.DMA,
        ],
    )(_sc_body)
    return run(idx, val, x)


def kernel(x, batch):
    batch_pad = jnp.pad(batch, (0, NPAD - N), constant_values=G).reshape(
        NB, 1, BR
    )
    keys, se = _tc_stage(x, batch_pad)
    keys_flat2d = jnp.pad(
        keys.reshape(-1), (0, KPAD - NPAD), constant_values=-jnp.inf
    ).reshape(KROWS, WCOLS)
    idx, val = _sel_stage(se, keys_flat2d)
    out = _sc_stage(idx, val, x)
    return out.reshape(G, K * D)
